# baseline (speedup 1.0000x reference)
import jax
import jax.numpy as jnp
from jax.experimental import pallas as pl

NEG_SLOPE = 0.2


def _layer_norm(x, g, b):
    mu = jnp.mean(x, axis=-1, keepdims=True)
    var = jnp.mean((x - mu) ** 2, axis=-1, keepdims=True)
    return (x - mu) / jnp.sqrt(var + 1e-5) * g + b


def _elu(x):
    return jnp.where(x > 0, x, jnp.expm1(x))


def _gcn_conv(x, src, dst, ew, W, b):
    n = x.shape[0]
    loop = jnp.arange(n, dtype=src.dtype)
    s = jnp.concatenate([src, loop])
    d = jnp.concatenate([dst, loop])
    w = jnp.concatenate([ew, jnp.ones((n,), x.dtype)])
    deg = jnp.zeros((n,), x.dtype).at[d].add(w)
    dinv = jnp.where(deg > 0, jax.lax.rsqrt(deg), 0.0)
    norm = dinv[s] * w * dinv[d]
    xw = x @ W
    out = jnp.zeros((n, xw.shape[1]), x.dtype).at[d].add(xw[s] * norm[:, None])
    return out + b


def _segment_softmax(logits, seg, num_segments):
    mx = jax.ops.segment_max(logits, seg, num_segments)
    mx = jnp.where(jnp.isfinite(mx), mx, 0.0)
    e = jnp.exp(logits - mx[seg])
    s = jax.ops.segment_sum(e, seg, num_segments)
    return e / (s[seg] + 1e-16)


def _gatv2_conv(x, src, dst, edge_attr, Wl, Wr, We, att, b):
    n = x.shape[0]
    loop = jnp.arange(n, dtype=src.dtype)
    s = jnp.concatenate([src, loop])
    d = jnp.concatenate([dst, loop])
    mean_ea = jnp.mean(edge_attr, axis=0, keepdims=True)
    ea = jnp.concatenate([edge_attr, jnp.broadcast_to(mean_ea, (n, edge_attr.shape[1]))], axis=0)
    xl = x @ Wl
    xr = x @ Wr
    m = xl[s] + xr[d] + ea @ We
    m = jnp.where(m > 0, m, NEG_SLOPE * m)
    logits = m @ att
    alpha = _segment_softmax(logits, d, n)
    out = jnp.zeros((n, xl.shape[1]), x.dtype).at[d].add(xl[s] * alpha[:, None])
    return out + b, alpha


def kernel(x, edge_index, edge_attr, params):
    src = edge_index[0]
    dst = edge_index[1]
    ew = edge_attr[:, 0]
    h = _gcn_conv(x, src, dst, ew, params['gcn0_W'], params['gcn0_b'])
    x1 = h + (x @ params['res0_W'] + params['res0_b'])
    x1 = _elu(_layer_norm(x1, params['ln0_g'], params['ln0_b']))
    h = _gcn_conv(x1, src, dst, ew, params['gcn1_W'], params['gcn1_b'])
    x2 = h + (x1 @ params['res1_W'] + params['res1_b'])
    x2 = _elu(_layer_norm(x2, params['ln1_g'], params['ln1_b']))
    h, alpha = _gatv2_conv(x2, src, dst, edge_attr, params['gat_Wl'], params['gat_Wr'],
                           params['gat_We'], params['gat_att'], params['gat_b'])
    x3 = h + (x2 @ params['res2_W'] + params['res2_b'])
    x3 = _elu(_layer_norm(x3, params['ln2_g'], params['ln2_b']))
    ht = jax.nn.relu(x3 @ params['ct1_W'] + params['ct1_b'])
    hc = jax.nn.relu(x3 @ params['cl1_W'] + params['cl1_b'])
    ht = jax.nn.log_softmax(ht @ params['ct2_W'] + params['ct2_b'], axis=-1)
    hc = jax.nn.log_softmax(hc @ params['cl2_W'] + params['cl2_b'], axis=-1)
    return jnp.concatenate([hc, ht], axis=1), alpha


# speedup vs baseline: 10.6073x; 10.6073x over previous
import functools

import jax
import jax.numpy as jnp
from jax import lax
from jax.experimental import pallas as pl
from jax.experimental.pallas import tpu as pltpu
from jax.experimental.pallas import tpu_sc as plsc

N = 100000
E = 3200000
DIM_H = 16
NEG_SLOPE = 0.2

NC = 2
NS = 16
NW = NC * NS
IR = 128
KC = 8
CH = KC * IR

Ep = 3211264
R = Ep // IR
RPW = R // NW
RPW2 = R // NW
Np = 100352
NPS = Np // NS

_mesh = plsc.VectorSubcoreMesh(core_axis_name="c", subcore_axis_name="s")


def _wid(c, s):
    return s * NC + c


@functools.partial(
    pl.kernel,
    out_type=jax.ShapeDtypeStruct((Ep, DIM_H), jnp.float32),
    mesh=_mesh,
    compiler_params=pltpu.CompilerParams(use_tc_tiling_on_sc=False),
    scratch_types=[
        pltpu.VMEM((KC, IR), jnp.int32),
        pltpu.VMEM((CH, DIM_H), jnp.float32),
        pltpu.SemaphoreType.DMA,
    ],
)
def _sc_gather_rows(table, idx2, out, idxv, rowsv, sem):
    c = lax.axis_index("c")
    s = lax.axis_index("s")
    base = _wid(c, s) * RPW

    def chunk(j, carry):
        r0 = base + j * KC
        pltpu.sync_copy(idx2.at[pl.ds(r0, KC)], idxv)
        descs = [
            pltpu.async_copy(table.at[idxv.at[k]],
                             rowsv.at[pl.ds(k * IR, IR)], sem)
            for k in range(KC)
        ]
        for d in descs:
            d.wait()
        pltpu.sync_copy(rowsv, out.at[pl.ds(r0 * IR, CH)])
        return carry

    lax.fori_loop(0, RPW // KC, chunk, 0)


@functools.partial(
    pl.kernel,
    out_type=jax.ShapeDtypeStruct((NC * Np, DIM_H), jnp.float32),
    mesh=_mesh,
    compiler_params=pltpu.CompilerParams(use_tc_tiling_on_sc=False),
    scratch_types=[
        pltpu.VMEM_SHARED((Np, DIM_H), jnp.float32),
        pltpu.VMEM((64, DIM_H), jnp.float32),
        pltpu.VMEM((KC, IR), jnp.int32),
        pltpu.VMEM((CH, DIM_H), jnp.float32),
        pltpu.SemaphoreType.DMA,
    ],
)
def _sc_scatter_add_rows(vals, idx2, out, acc, zb, idxv, valsv, sem):
    c = lax.axis_index("c")
    s = lax.axis_index("s")
    for k in range(64):
        zb[k] = jnp.zeros((DIM_H,), jnp.float32)
    for t in range(NPS // 64):
        pltpu.sync_copy(zb, acc.at[pl.ds(s * NPS + t * 64, 64)])
    plsc.subcore_barrier()

    base = (c * NS + s) * RPW

    def chunk(j, carry):
        r0 = base + j * KC
        pltpu.sync_copy(idx2.at[pl.ds(r0, KC)], idxv)
        pltpu.sync_copy(vals.at[pl.ds(r0 * IR, CH)], valsv)
        descs = [
            pltpu.async_copy(valsv.at[pl.ds(k * IR, IR)],
                             acc.at[idxv.at[k]], sem, add=True)
            for k in range(KC)
        ]
        for d in descs:
            d.wait()
        return carry

    lax.fori_loop(0, RPW // KC, chunk, 0)
    plsc.subcore_barrier()
    pltpu.sync_copy(acc.at[pl.ds(s * NPS, NPS)],
                    out.at[pl.ds(c * Np + s * NPS, NPS)])


@functools.partial(
    pl.kernel,
    out_type=jax.ShapeDtypeStruct((NC * Np,), jnp.float32),
    mesh=_mesh,
    compiler_params=pltpu.CompilerParams(use_tc_tiling_on_sc=False),
    scratch_types=[
        pltpu.VMEM_SHARED((Np,), jnp.float32),
        pltpu.VMEM((784,), jnp.float32),
        pltpu.VMEM((KC, IR), jnp.int32),
        pltpu.VMEM((KC, IR), jnp.float32),
        pltpu.SemaphoreType.DMA,
    ],
)
def _sc_scatter_add_scalar(vals2, idx2, out, acc, zb, idxv, valsv, sem):
    c = lax.axis_index("c")
    s = lax.axis_index("s")
    for k in range(784 // 16):
        zb[pl.ds(k * 16, 16)] = jnp.zeros((16,), jnp.float32)
    for t in range(NPS // 784):
        pltpu.sync_copy(zb, acc.at[pl.ds(s * NPS + t * 784, 784)])
    plsc.subcore_barrier()

    base = (c * NS + s) * RPW

    def chunk(j, carry):
        r0 = base + j * KC
        pltpu.sync_copy(idx2.at[pl.ds(r0, KC)], idxv)
        pltpu.sync_copy(vals2.at[pl.ds(r0, KC)], valsv)
        descs = [
            pltpu.async_copy(valsv.at[k], acc.at[idxv.at[k]], sem, add=True)
            for k in range(KC)
        ]
        for d in descs:
            d.wait()
        return carry

    lax.fori_loop(0, RPW // KC, chunk, 0)
    plsc.subcore_barrier()
    pltpu.sync_copy(acc.at[pl.ds(s * NPS, NPS)],
                    out.at[pl.ds(c * Np + s * NPS, NPS)])


@functools.partial(
    pl.kernel,
    out_type=jax.ShapeDtypeStruct((Ep,), jnp.float32),
    mesh=_mesh,
    compiler_params=pltpu.CompilerParams(use_tc_tiling_on_sc=False,
                                         needs_layout_passes=False),
    scratch_types=[
        pltpu.VMEM((Np,), jnp.float32),
        pltpu.VMEM((CH,), jnp.int32),
        pltpu.VMEM((CH,), jnp.float32),
        pltpu.SemaphoreType.DMA,
    ],
)
def _sc_gather_scalar(table, idx, out, tabv, idxv, outv, sem):
    c = lax.axis_index("c")
    s = lax.axis_index("s")
    base = _wid(c, s) * RPW * IR
    pltpu.sync_copy(table, tabv)

    def chunk(j, carry):
        e0 = base + j * CH
        pltpu.sync_copy(idx.at[pl.ds(e0, CH)], idxv)

        def veci(t, carry2):
            o = pl.multiple_of(t * 16, 16)
            iv = idxv[pl.ds(o, 16)]
            outv[pl.ds(o, 16)] = plsc.load_gather(tabv, [iv])
            return carry2

        lax.fori_loop(0, CH // 16, veci, 0)
        pltpu.sync_copy(outv, out.at[pl.ds(e0, CH)])
        return carry

    lax.fori_loop(0, RPW * IR // CH, chunk, 0)


def _layer_norm(x, g, b):
    mu = jnp.mean(x, axis=-1, keepdims=True)
    var = jnp.mean((x - mu) ** 2, axis=-1, keepdims=True)
    return (x - mu) / jnp.sqrt(var + 1e-5) * g + b


def _elu(x):
    return jnp.where(x > 0, x, jnp.expm1(x))


def _pad_rows(t):
    return jnp.pad(t, ((0, Np - N), (0, 0)))


def kernel(x, edge_index, edge_attr, params):
    src = edge_index[0]
    dst = edge_index[1]
    ew = edge_attr[:, 0]

    srcp = jnp.pad(src, (0, Ep - E))
    dstp = jnp.pad(dst, (0, Ep - E))
    ewp = jnp.pad(ew, (0, Ep - E))
    src2 = srcp.reshape(R, IR)
    dst2 = dstp.reshape(R, IR)
    ew2 = ewp.reshape(R, IR)
    valid = jnp.arange(Ep, dtype=jnp.int32) < E

    degp = _sc_scatter_add_scalar(ew2, dst2)
    deg = degp[:N] + degp[Np:Np + N] + 1.0
    dinv = lax.rsqrt(deg)

    def gcn(xin, W, b):
        xw = xin @ W
        u = xw * dinv[:, None]
        g = _sc_gather_rows(_pad_rows(u), src2)
        vals = g * ewp[:, None]
        hp = _sc_scatter_add_rows(vals, dst2)
        h = (hp[:N] + hp[Np:Np + N]) * dinv[:, None] + xw * (dinv * dinv)[:, None]
        return h + b

    h = gcn(x, params['gcn0_W'], params['gcn0_b'])
    x1 = _elu(_layer_norm(h + x @ params['res0_W'] + params['res0_b'],
                          params['ln0_g'], params['ln0_b']))
    h = gcn(x1, params['gcn1_W'], params['gcn1_b'])
    x2 = _elu(_layer_norm(h + x1 @ params['res1_W'] + params['res1_b'],
                          params['ln1_g'], params['ln1_b']))

    xl = x2 @ params['gat_Wl']
    xr = x2 @ params['gat_Wr']
    We_row = params['gat_We'][0]
    att = params['gat_att']
    gl = _sc_gather_rows(_pad_rows(xl), src2)
    gr = _sc_gather_rows(_pad_rows(xr), dst2)
    z = gl + gr + ewp[:, None] * We_row
    m = jnp.where(z > 0, z, NEG_SLOPE * z)
    logits = m @ att

    mean_ea = jnp.mean(ew)
    z_self = xl + xr + mean_ea * We_row
    m_self = jnp.where(z_self > 0, z_self, NEG_SLOPE * z_self)
    logit_self = m_self @ att

    M = jnp.maximum(jnp.max(jnp.where(valid, logits, -jnp.inf)),
                    jnp.max(logit_self))
    e = jnp.where(valid, jnp.exp(logits - M), 0.0)
    e_self = jnp.exp(logit_self - M)

    ssump = _sc_scatter_add_scalar(e.reshape(R, IR), dst2)
    ssum = ssump[:N] + ssump[Np:Np + N] + e_self
    rr = 1.0 / (ssum + 1e-16)

    rg = _sc_gather_scalar(jnp.pad(rr, (0, Np - N)), dstp)
    alpha_e = e * rg
    alpha_self = e_self * rr

    hp = _sc_scatter_add_rows(gl * alpha_e[:, None], dst2)
    hgat = (hp[:N] + hp[Np:Np + N]) + xl * alpha_self[:, None] + params['gat_b']

    x3 = _elu(_layer_norm(hgat + x2 @ params['res2_W'] + params['res2_b'],
                          params['ln2_g'], params['ln2_b']))
    ht = jax.nn.relu(x3 @ params['ct1_W'] + params['ct1_b'])
    hc = jax.nn.relu(x3 @ params['cl1_W'] + params['cl1_b'])
    ht = jax.nn.log_softmax(ht @ params['ct2_W'] + params['ct2_b'], axis=-1)
    hc = jax.nn.log_softmax(hc @ params['cl2_W'] + params['cl2_b'], axis=-1)
    out = jnp.concatenate([hc, ht], axis=1)
    alpha = jnp.concatenate([alpha_e[:E], alpha_self])
    return out, alpha


# speedup vs baseline: 26.3185x; 2.4812x over previous
import functools

import jax
import jax.numpy as jnp
from jax import lax
from jax.experimental import pallas as pl
from jax.experimental.pallas import tpu as pltpu
from jax.experimental.pallas import tpu_sc as plsc

N = 100000
E = 3200000
DIM_H = 16
NEG_SLOPE = 0.2

NC = 2
NS = 16
NW = NC * NS
IR = 128
KC = 8
CH = KC * IR

Ep = 3211264
R = Ep // IR
RPW = R // NW
Np = 100352
NPS = Np // NS

_mesh = plsc.VectorSubcoreMesh(core_axis_name="c", subcore_axis_name="s")
_params = pltpu.CompilerParams(use_tc_tiling_on_sc=False,
                               needs_layout_passes=False)


def _zero_acc_rows(acc, zb, s):
    for k in range(64):
        zb[k] = jnp.zeros((DIM_H,), jnp.float32)
    for t in range(NPS // 64):
        pltpu.sync_copy(zb, acc.at[pl.ds(s * NPS + t * 64, 64)])


@functools.partial(
    pl.kernel,
    out_type=jax.ShapeDtypeStruct((NC * Np, DIM_H), jnp.float32),
    mesh=_mesh,
    compiler_params=_params,
    scratch_types=[
        pltpu.VMEM_SHARED((Np, DIM_H), jnp.float32),
        pltpu.VMEM((64, DIM_H), jnp.float32),
        pltpu.VMEM((KC, IR), jnp.int32),
        pltpu.VMEM((KC, IR), jnp.int32),
        pltpu.VMEM((CH,), jnp.float32),
        pltpu.VMEM((CH, DIM_H), jnp.float32),
        pltpu.SemaphoreType.DMA,
    ],
)
def _sc_conv(table, src2, dst2, scl, out, acc, zb, srcv, dstv, sclv, rowsv,
             sem):
    c = lax.axis_index("c")
    s = lax.axis_index("s")
    _zero_acc_rows(acc, zb, s)
    plsc.subcore_barrier()

    base = (c * NS + s) * RPW

    def chunk(j, carry):
        r0 = base + j * KC
        pltpu.sync_copy(src2.at[pl.ds(r0, KC)], srcv)
        pltpu.sync_copy(dst2.at[pl.ds(r0, KC)], dstv)
        pltpu.sync_copy(scl.at[pl.ds(r0 * IR, CH)], sclv)
        gd = [
            pltpu.async_copy(table.at[srcv.at[k]],
                             rowsv.at[pl.ds(k * IR, IR)], sem)
            for k in range(KC)
        ]
        for d in gd:
            d.wait()

        def scale(i, carry2):
            spl = plsc.load_gather(sclv, [jnp.full((16,), i, jnp.int32)])
            rowsv[i] = rowsv[i] * spl
            return carry2

        lax.fori_loop(0, CH, scale, 0)
        sd = [
            pltpu.async_copy(rowsv.at[pl.ds(k * IR, IR)],
                             acc.at[dstv.at[k]], sem, add=True)
            for k in range(KC)
        ]
        for d in sd:
            d.wait()
        return carry

    lax.fori_loop(0, RPW // KC, chunk, 0)
    plsc.subcore_barrier()
    pltpu.sync_copy(acc.at[pl.ds(s * NPS, NPS)],
                    out.at[pl.ds(c * Np + s * NPS, NPS)])


@functools.partial(
    pl.kernel,
    out_type=jax.ShapeDtypeStruct((Ep,), jnp.float32),
    mesh=_mesh,
    compiler_params=_params,
    scratch_types=[
        pltpu.VMEM((KC, IR), jnp.int32),
        pltpu.VMEM((KC, IR), jnp.int32),
        pltpu.VMEM((CH,), jnp.float32),
        pltpu.VMEM((CH, DIM_H), jnp.float32),
        pltpu.VMEM((CH, DIM_H), jnp.float32),
        pltpu.VMEM((CH,), jnp.float32),
        pltpu.VMEM((16,), jnp.float32),
        pltpu.VMEM((16,), jnp.float32),
        pltpu.SemaphoreType.DMA,
    ],
)
def _sc_gat_logits(xl, xr, src2, dst2, ew, we, att, out, srcv, dstv, ewv,
                   rlv, rrv, outv, wev, attv, sem):
    c = lax.axis_index("c")
    s = lax.axis_index("s")
    base = (c * NS + s) * RPW
    pltpu.sync_copy(we, wev)
    pltpu.sync_copy(att, attv)
    lane = lax.iota(jnp.int32, 16)

    def chunk(j, carry):
        r0 = base + j * KC
        pltpu.sync_copy(src2.at[pl.ds(r0, KC)], srcv)
        pltpu.sync_copy(dst2.at[pl.ds(r0, KC)], dstv)
        pltpu.sync_copy(ew.at[pl.ds(r0 * IR, CH)], ewv)
        gd = [
            pltpu.async_copy(xl.at[srcv.at[k]],
                             rlv.at[pl.ds(k * IR, IR)], sem)
            for k in range(KC)
        ] + [
            pltpu.async_copy(xr.at[dstv.at[k]],
                             rrv.at[pl.ds(k * IR, IR)], sem)
            for k in range(KC)
        ]
        for d in gd:
            d.wait()
        wv = wev[...]
        av = attv[...]

        def group(g, carry2):
            acc = jnp.zeros((16,), jnp.float32)
            for t in range(16):
                i = g * 16 + t
                spl = plsc.load_gather(ewv, [jnp.full((16,), i, jnp.int32)])
                z = rlv[i] + rrv[i] + wv * spl
                m = jnp.maximum(z, 0.0) + NEG_SLOPE * jnp.minimum(z, 0.0)
                sc = jnp.sum(m * av)
                acc = jnp.where(lane == t, sc, acc)
            o = pl.multiple_of(g * 16, 16)
            outv[pl.ds(o, 16)] = acc
            return carry2

        lax.fori_loop(0, CH // 16, group, 0)
        pltpu.sync_copy(outv, out.at[pl.ds(r0 * IR, CH)])
        return carry

    lax.fori_loop(0, RPW // KC, chunk, 0)


@functools.partial(
    pl.kernel,
    out_type=jax.ShapeDtypeStruct((NC * Np,), jnp.float32),
    mesh=_mesh,
    compiler_params=_params,
    scratch_types=[
        pltpu.VMEM_SHARED((Np,), jnp.float32),
        pltpu.VMEM((784,), jnp.float32),
        pltpu.VMEM((KC, IR), jnp.int32),
        pltpu.VMEM((KC, IR), jnp.float32),
        pltpu.SemaphoreType.DMA,
    ],
)
def _sc_scatter_add_scalar(vals2, idx2, out, acc, zb, idxv, valsv, sem):
    c = lax.axis_index("c")
    s = lax.axis_index("s")
    for k in range(784 // 16):
        zb[pl.ds(k * 16, 16)] = jnp.zeros((16,), jnp.float32)
    for t in range(NPS // 784):
        pltpu.sync_copy(zb, acc.at[pl.ds(s * NPS + t * 784, 784)])
    plsc.subcore_barrier()

    base = (c * NS + s) * RPW

    def chunk(j, carry):
        r0 = base + j * KC
        pltpu.sync_copy(idx2.at[pl.ds(r0, KC)], idxv)
        pltpu.sync_copy(vals2.at[pl.ds(r0, KC)], valsv)
        descs = [
            pltpu.async_copy(valsv.at[k], acc.at[idxv.at[k]], sem, add=True)
            for k in range(KC)
        ]
        for d in descs:
            d.wait()
        return carry

    lax.fori_loop(0, RPW // KC, chunk, 0)
    plsc.subcore_barrier()
    pltpu.sync_copy(acc.at[pl.ds(s * NPS, NPS)],
                    out.at[pl.ds(c * Np + s * NPS, NPS)])


@functools.partial(
    pl.kernel,
    out_type=jax.ShapeDtypeStruct((Ep,), jnp.float32),
    mesh=_mesh,
    compiler_params=_params,
    scratch_types=[
        pltpu.VMEM((Np,), jnp.float32),
        pltpu.VMEM((CH,), jnp.int32),
        pltpu.VMEM((CH,), jnp.float32),
        pltpu.SemaphoreType.DMA,
    ],
)
def _sc_gather_scalar(table, idx, out, tabv, idxv, outv, sem):
    c = lax.axis_index("c")
    s = lax.axis_index("s")
    base = (s * NC + c) * RPW * IR
    pltpu.sync_copy(table, tabv)

    def chunk(j, carry):
        e0 = base + j * CH
        pltpu.sync_copy(idx.at[pl.ds(e0, CH)], idxv)

        def veci(t, carry2):
            o = pl.multiple_of(t * 16, 16)
            iv = idxv[pl.ds(o, 16)]
            outv[pl.ds(o, 16)] = plsc.load_gather(tabv, [iv])
            return carry2

        lax.fori_loop(0, CH // 16, veci, 0)
        pltpu.sync_copy(outv, out.at[pl.ds(e0, CH)])
        return carry

    lax.fori_loop(0, RPW * IR // CH, chunk, 0)


def _layer_norm(x, g, b):
    mu = jnp.mean(x, axis=-1, keepdims=True)
    var = jnp.mean((x - mu) ** 2, axis=-1, keepdims=True)
    return (x - mu) / jnp.sqrt(var + 1e-5) * g + b


def _elu(x):
    return jnp.where(x > 0, x, jnp.expm1(x))


def _pad_rows(t):
    return jnp.pad(t, ((0, Np - N), (0, 0)))


def kernel(x, edge_index, edge_attr, params):
    src = edge_index[0]
    dst = edge_index[1]
    ew = edge_attr[:, 0]

    srcp = jnp.pad(src, (0, Ep - E))
    dstp = jnp.pad(dst, (0, Ep - E))
    ewp = jnp.pad(ew, (0, Ep - E))
    src2 = srcp.reshape(R, IR)
    dst2 = dstp.reshape(R, IR)
    ew2 = ewp.reshape(R, IR)
    valid = jnp.arange(Ep, dtype=jnp.int32) < E

    degp = _sc_scatter_add_scalar(ew2, dst2)
    deg = degp[:N] + degp[Np:Np + N] + 1.0
    dinv = lax.rsqrt(deg)

    def gcn(xin, W, b):
        xw = xin @ W
        u = xw * dinv[:, None]
        hp = _sc_conv(_pad_rows(u), src2, dst2, ewp)
        h = (hp[:N] + hp[Np:Np + N]) * dinv[:, None] + xw * (dinv * dinv)[:, None]
        return h + b

    h = gcn(x, params['gcn0_W'], params['gcn0_b'])
    x1 = _elu(_layer_norm(h + x @ params['res0_W'] + params['res0_b'],
                          params['ln0_g'], params['ln0_b']))
    h = gcn(x1, params['gcn1_W'], params['gcn1_b'])
    x2 = _elu(_layer_norm(h + x1 @ params['res1_W'] + params['res1_b'],
                          params['ln1_g'], params['ln1_b']))

    xl = x2 @ params['gat_Wl']
    xr = x2 @ params['gat_Wr']
    We_row = params['gat_We'][0]
    att = params['gat_att']
    logits = _sc_gat_logits(_pad_rows(xl), _pad_rows(xr), src2, dst2, ewp,
                            We_row, att)

    mean_ea = jnp.mean(ew)
    z_self = xl + xr + mean_ea * We_row
    m_self = jnp.where(z_self > 0, z_self, NEG_SLOPE * z_self)
    logit_self = m_self @ att

    M = jnp.maximum(jnp.max(jnp.where(valid, logits, -jnp.inf)),
                    jnp.max(logit_self))
    e = jnp.where(valid, jnp.exp(logits - M), 0.0)
    e_self = jnp.exp(logit_self - M)

    ssump = _sc_scatter_add_scalar(e.reshape(R, IR), dst2)
    ssum = ssump[:N] + ssump[Np:Np + N] + e_self
    rr = 1.0 / (ssum + 1e-16)

    rg = _sc_gather_scalar(jnp.pad(rr, (0, Np - N)), dstp)
    alpha_e = e * rg
    alpha_self = e_self * rr

    hp = _sc_conv(_pad_rows(xl), src2, dst2, alpha_e)
    hgat = (hp[:N] + hp[Np:Np + N]) + xl * alpha_self[:, None] + params['gat_b']

    x3 = _elu(_layer_norm(hgat + x2 @ params['res2_W'] + params['res2_b'],
                          params['ln2_g'], params['ln2_b']))
    ht = jax.nn.relu(x3 @ params['ct1_W'] + params['ct1_b'])
    hc = jax.nn.relu(x3 @ params['cl1_W'] + params['cl1_b'])
    ht = jax.nn.log_softmax(ht @ params['ct2_W'] + params['ct2_b'], axis=-1)
    hc = jax.nn.log_softmax(hc @ params['cl2_W'] + params['cl2_b'], axis=-1)
    out = jnp.concatenate([hc, ht], axis=1)
    alpha = jnp.concatenate([alpha_e[:E], alpha_self])
    return out, alpha


# speedup vs baseline: 30.1333x; 1.1450x over previous
import functools

import jax
import jax.numpy as jnp
from jax import lax
from jax.experimental import pallas as pl
from jax.experimental.pallas import tpu as pltpu
from jax.experimental.pallas import tpu_sc as plsc

N = 100000
E = 3200000
DIM_H = 16
NEG_SLOPE = 0.2

NC = 2
NS = 16
NW = NC * NS
IR = 128
KC = 8
CH = KC * IR
KCC = 4
CHC = KCC * IR

Ep = 3211264
R = Ep // IR
RPW = R // NW
Np = 100352
NPS = Np // NS

_mesh = plsc.VectorSubcoreMesh(core_axis_name="c", subcore_axis_name="s")
_params = pltpu.CompilerParams(use_tc_tiling_on_sc=False,
                               needs_layout_passes=False)


def _zero_acc_rows(acc, zb, s):
    for k in range(64):
        zb[k] = jnp.zeros((DIM_H,), jnp.float32)
    for t in range(NPS // 64):
        pltpu.sync_copy(zb, acc.at[pl.ds(s * NPS + t * 64, 64)])


@functools.partial(
    pl.kernel,
    out_type=jax.ShapeDtypeStruct((NC * Np, DIM_H), jnp.float32),
    mesh=_mesh,
    compiler_params=_params,
    scratch_types=[
        pltpu.VMEM_SHARED((Np, DIM_H), jnp.float32),
        pltpu.VMEM((64, DIM_H), jnp.float32),
        pltpu.VMEM((2, KCC, IR), jnp.int32),
        pltpu.VMEM((2, KCC, IR), jnp.int32),
        pltpu.VMEM((2, CHC), jnp.float32),
        pltpu.VMEM((2, CHC, DIM_H), jnp.float32),
        pltpu.SemaphoreType.DMA,
        pltpu.SemaphoreType.DMA,
    ],
)
def _sc_conv(table, src2, dst2, scl, out, acc, zb, srcv, dstv, sclv, rowsv,
             semg, sems):
    c = lax.axis_index("c")
    s = lax.axis_index("s")
    _zero_acc_rows(acc, zb, s)
    plsc.subcore_barrier()

    base = (c * NS + s) * RPW
    NCH = RPW // KCC

    def load_idx(j, p):
        r0 = base + j * KCC
        pltpu.sync_copy(src2.at[pl.ds(r0, KCC)], srcv.at[p])
        pltpu.sync_copy(dst2.at[pl.ds(r0, KCC)], dstv.at[p])
        pltpu.sync_copy(scl.at[pl.ds(r0 * IR, CHC)], sclv.at[p])

    def fire_gathers(p):
        for k in range(KCC):
            pltpu.async_copy(table.at[srcv.at[p].at[k]],
                             rowsv.at[p].at[pl.ds(k * IR, IR)], semg)

    def wait_gathers(p):
        for k in range(KCC):
            pltpu.make_async_copy(table.at[srcv.at[p].at[k]],
                                  rowsv.at[p].at[pl.ds(k * IR, IR)],
                                  semg).wait()

    def fire_scatters(p):
        for k in range(KCC):
            pltpu.async_copy(rowsv.at[p].at[pl.ds(k * IR, IR)],
                             acc.at[dstv.at[p].at[k]], sems, add=True)

    def wait_scatters(p):
        for k in range(KCC):
            pltpu.make_async_copy(rowsv.at[p].at[pl.ds(k * IR, IR)],
                                  acc.at[dstv.at[p].at[k]], sems).wait()

    def scale_rows(p):
        def scale(i, carry2):
            spl = plsc.load_gather(sclv.at[p], [jnp.full((16,), i, jnp.int32)])
            rowsv[p, i] = rowsv[p, i] * spl
            return carry2

        lax.fori_loop(0, CHC, scale, 0, unroll=8)

    load_idx(0, 0)
    fire_gathers(0)

    def pair(jj, carry):
        a = 2 * jj

        @pl.when(jj > 0)
        def _():
            wait_scatters(1)

        load_idx(a + 1, 1)
        fire_gathers(1)
        wait_gathers(0)
        scale_rows(0)
        fire_scatters(0)
        wait_scatters(0)
        load_idx(jnp.minimum(a + 2, NCH - 1), 0)
        fire_gathers(0)
        wait_gathers(1)
        scale_rows(1)
        fire_scatters(1)
        return carry

    lax.fori_loop(0, NCH // 2, pair, 0)
    wait_gathers(0)
    wait_scatters(1)
    plsc.subcore_barrier()
    pltpu.sync_copy(acc.at[pl.ds(s * NPS, NPS)],
                    out.at[pl.ds(c * Np + s * NPS, NPS)])


@functools.partial(
    pl.kernel,
    out_type=jax.ShapeDtypeStruct((Ep,), jnp.float32),
    mesh=_mesh,
    compiler_params=_params,
    scratch_types=[
        pltpu.VMEM((2, KC, IR), jnp.int32),
        pltpu.VMEM((2, KC, IR), jnp.int32),
        pltpu.VMEM((2, CH), jnp.float32),
        pltpu.VMEM((2, CH, DIM_H), jnp.float32),
        pltpu.VMEM((2, CH, DIM_H), jnp.float32),
        pltpu.VMEM((2, CH), jnp.float32),
        pltpu.VMEM((16,), jnp.float32),
        pltpu.VMEM((16,), jnp.float32),
        pltpu.SemaphoreType.DMA,
    ],
)
def _sc_gat_logits(xl, xr, src2, dst2, ew, we, att, out, srcv, dstv, ewv,
                   rlv, rrv, outv, wev, attv, sem):
    c = lax.axis_index("c")
    s = lax.axis_index("s")
    base = (c * NS + s) * RPW
    NCH = RPW // KC
    pltpu.sync_copy(we, wev)
    pltpu.sync_copy(att, attv)
    lane = lax.iota(jnp.int32, 16)
    wv = wev[...]
    av = attv[...]

    def load_idx(j, p):
        r0 = base + j * KC
        pltpu.sync_copy(src2.at[pl.ds(r0, KC)], srcv.at[p])
        pltpu.sync_copy(dst2.at[pl.ds(r0, KC)], dstv.at[p])
        pltpu.sync_copy(ew.at[pl.ds(r0 * IR, CH)], ewv.at[p])

    def fire_gathers(p):
        for k in range(KC):
            pltpu.async_copy(xl.at[srcv.at[p].at[k]],
                             rlv.at[p].at[pl.ds(k * IR, IR)], sem)
            pltpu.async_copy(xr.at[dstv.at[p].at[k]],
                             rrv.at[p].at[pl.ds(k * IR, IR)], sem)

    def wait_gathers(p):
        for k in range(KC):
            pltpu.make_async_copy(xl.at[srcv.at[p].at[k]],
                                  rlv.at[p].at[pl.ds(k * IR, IR)], sem).wait()
            pltpu.make_async_copy(xr.at[dstv.at[p].at[k]],
                                  rrv.at[p].at[pl.ds(k * IR, IR)], sem).wait()

    def compute(j, p):
        def group(g, carry2):
            acc = jnp.zeros((16,), jnp.float32)
            for t in range(16):
                i = g * 16 + t
                spl = plsc.load_gather(ewv.at[p],
                                       [jnp.full((16,), i, jnp.int32)])
                z = rlv[p, i] + rrv[p, i] + wv * spl
                m = jnp.maximum(z, 0.0) + NEG_SLOPE * jnp.minimum(z, 0.0)
                sc = jnp.sum(m * av)
                acc = jnp.where(lane == t, sc, acc)
            o = pl.multiple_of(g * 16, 16)
            outv[p, pl.ds(o, 16)] = acc
            return carry2

        lax.fori_loop(0, CH // 16, group, 0, unroll=2)
        r0 = base + j * KC
        pltpu.sync_copy(outv.at[p], out.at[pl.ds(r0 * IR, CH)])

    load_idx(0, 0)
    fire_gathers(0)

    def pair(jj, carry):
        a = 2 * jj
        load_idx(a + 1, 1)
        fire_gathers(1)
        wait_gathers(0)
        compute(a, 0)
        load_idx(jnp.minimum(a + 2, NCH - 1), 0)
        fire_gathers(0)
        wait_gathers(1)
        compute(a + 1, 1)
        return carry

    lax.fori_loop(0, NCH // 2, pair, 0)
    wait_gathers(0)


@functools.partial(
    pl.kernel,
    out_type=jax.ShapeDtypeStruct((NC * Np,), jnp.float32),
    mesh=_mesh,
    compiler_params=_params,
    scratch_types=[
        pltpu.VMEM_SHARED((Np,), jnp.float32),
        pltpu.VMEM((784,), jnp.float32),
        pltpu.VMEM((KC, IR), jnp.int32),
        pltpu.VMEM((KC, IR), jnp.float32),
        pltpu.SemaphoreType.DMA,
    ],
)
def _sc_scatter_add_scalar(vals2, idx2, out, acc, zb, idxv, valsv, sem):
    c = lax.axis_index("c")
    s = lax.axis_index("s")
    for k in range(784 // 16):
        zb[pl.ds(k * 16, 16)] = jnp.zeros((16,), jnp.float32)
    for t in range(NPS // 784):
        pltpu.sync_copy(zb, acc.at[pl.ds(s * NPS + t * 784, 784)])
    plsc.subcore_barrier()

    base = (c * NS + s) * RPW

    def chunk(j, carry):
        r0 = base + j * KC
        pltpu.sync_copy(idx2.at[pl.ds(r0, KC)], idxv)
        pltpu.sync_copy(vals2.at[pl.ds(r0, KC)], valsv)
        descs = [
            pltpu.async_copy(valsv.at[k], acc.at[idxv.at[k]], sem, add=True)
            for k in range(KC)
        ]
        for d in descs:
            d.wait()
        return carry

    lax.fori_loop(0, RPW // KC, chunk, 0)
    plsc.subcore_barrier()
    pltpu.sync_copy(acc.at[pl.ds(s * NPS, NPS)],
                    out.at[pl.ds(c * Np + s * NPS, NPS)])


@functools.partial(
    pl.kernel,
    out_type=jax.ShapeDtypeStruct((Ep,), jnp.float32),
    mesh=_mesh,
    compiler_params=_params,
    scratch_types=[
        pltpu.VMEM((Np,), jnp.float32),
        pltpu.VMEM((CH,), jnp.int32),
        pltpu.VMEM((CH,), jnp.float32),
        pltpu.SemaphoreType.DMA,
    ],
)
def _sc_gather_scalar(table, idx, out, tabv, idxv, outv, sem):
    c = lax.axis_index("c")
    s = lax.axis_index("s")
    base = (s * NC + c) * RPW * IR
    pltpu.sync_copy(table, tabv)

    def chunk(j, carry):
        e0 = base + j * CH
        pltpu.sync_copy(idx.at[pl.ds(e0, CH)], idxv)

        def veci(t, carry2):
            o = pl.multiple_of(t * 16, 16)
            iv = idxv[pl.ds(o, 16)]
            outv[pl.ds(o, 16)] = plsc.load_gather(tabv, [iv])
            return carry2

        lax.fori_loop(0, CH // 16, veci, 0)
        pltpu.sync_copy(outv, out.at[pl.ds(e0, CH)])
        return carry

    lax.fori_loop(0, RPW * IR // CH, chunk, 0)


def _layer_norm(x, g, b):
    mu = jnp.mean(x, axis=-1, keepdims=True)
    var = jnp.mean((x - mu) ** 2, axis=-1, keepdims=True)
    return (x - mu) / jnp.sqrt(var + 1e-5) * g + b


def _elu(x):
    return jnp.where(x > 0, x, jnp.expm1(x))


def _pad_rows(t):
    return jnp.pad(t, ((0, Np - N), (0, 0)))


def kernel(x, edge_index, edge_attr, params):
    src = edge_index[0]
    dst = edge_index[1]
    ew = edge_attr[:, 0]

    srcp = jnp.pad(src, (0, Ep - E))
    dstp = jnp.pad(dst, (0, Ep - E))
    ewp = jnp.pad(ew, (0, Ep - E))
    src2 = srcp.reshape(R, IR)
    dst2 = dstp.reshape(R, IR)
    ew2 = ewp.reshape(R, IR)
    valid = jnp.arange(Ep, dtype=jnp.int32) < E

    degp = _sc_scatter_add_scalar(ew2, dst2)
    deg = degp[:N] + degp[Np:Np + N] + 1.0
    dinv = lax.rsqrt(deg)

    def gcn(xin, W, b):
        xw = xin @ W
        u = xw * dinv[:, None]
        hp = _sc_conv(_pad_rows(u), src2, dst2, ewp)
        h = (hp[:N] + hp[Np:Np + N]) * dinv[:, None] + xw * (dinv * dinv)[:, None]
        return h + b

    h = gcn(x, params['gcn0_W'], params['gcn0_b'])
    x1 = _elu(_layer_norm(h + x @ params['res0_W'] + params['res0_b'],
                          params['ln0_g'], params['ln0_b']))
    h = gcn(x1, params['gcn1_W'], params['gcn1_b'])
    x2 = _elu(_layer_norm(h + x1 @ params['res1_W'] + params['res1_b'],
                          params['ln1_g'], params['ln1_b']))

    xl = x2 @ params['gat_Wl']
    xr = x2 @ params['gat_Wr']
    We_row = params['gat_We'][0]
    att = params['gat_att']
    logits = _sc_gat_logits(_pad_rows(xl), _pad_rows(xr), src2, dst2, ewp,
                            We_row, att)

    mean_ea = jnp.mean(ew)
    z_self = xl + xr + mean_ea * We_row
    m_self = jnp.where(z_self > 0, z_self, NEG_SLOPE * z_self)
    logit_self = m_self @ att

    M = jnp.maximum(jnp.max(jnp.where(valid, logits, -jnp.inf)),
                    jnp.max(logit_self))
    e = jnp.where(valid, jnp.exp(logits - M), 0.0)
    e_self = jnp.exp(logit_self - M)

    ssump = _sc_scatter_add_scalar(e.reshape(R, IR), dst2)
    ssum = ssump[:N] + ssump[Np:Np + N] + e_self
    rr = 1.0 / (ssum + 1e-16)

    rg = _sc_gather_scalar(jnp.pad(rr, (0, Np - N)), dstp)
    alpha_e = e * rg
    alpha_self = e_self * rr

    hp = _sc_conv(_pad_rows(xl), src2, dst2, alpha_e)
    hgat = (hp[:N] + hp[Np:Np + N]) + xl * alpha_self[:, None] + params['gat_b']

    x3 = _elu(_layer_norm(hgat + x2 @ params['res2_W'] + params['res2_b'],
                          params['ln2_g'], params['ln2_b']))
    ht = jax.nn.relu(x3 @ params['ct1_W'] + params['ct1_b'])
    hc = jax.nn.relu(x3 @ params['cl1_W'] + params['cl1_b'])
    ht = jax.nn.log_softmax(ht @ params['ct2_W'] + params['ct2_b'], axis=-1)
    hc = jax.nn.log_softmax(hc @ params['cl2_W'] + params['cl2_b'], axis=-1)
    out = jnp.concatenate([hc, ht], axis=1)
    alpha = jnp.concatenate([alpha_e[:E], alpha_self])
    return out, alpha
